# baseline (speedup 1.0000x reference)
import functools

import jax
import jax.numpy as jnp
from jax import lax
from jax.experimental import pallas as pl
from jax.experimental.pallas import tpu as pltpu
from jax.experimental.pallas import tpu_sc as plsc

N = 10000
E = 320000
S = 200000
D = 128

NC = 2
NS = 16
NW = NC * NS

NP = 10240
ROWS_T = NP // NS

CHUNK = 512
ECH = 20
EPT = ECH * CHUNK
EP = EPT * NW

PCHUNK = 128
SCH = 49
SPT = SCH * PCHUNK
SP = SPT * NW


def _sc_mesh():
    return plsc.VectorSubcoreMesh(core_axis_name="c", subcore_axis_name="s",
                                  num_cores=NC, num_subcores=NS)


_SC_PARAMS = pltpu.CompilerParams(use_tc_tiling_on_sc=False)


@functools.partial(
    pl.kernel,
    out_type=jax.ShapeDtypeStruct((NC * NP,), jnp.float32),
    mesh=_sc_mesh(),
    compiler_params=_SC_PARAMS,
    scratch_types=[
        pltpu.VMEM((ECH, CHUNK), jnp.int32),
        pltpu.VMEM((CHUNK,), jnp.float32),
        pltpu.VMEM((ROWS_T,), jnp.float32),
        pltpu.VMEM_SHARED((NP,), jnp.float32),
        pltpu.SemaphoreType.DMA,
    ],
)
def _hist_kernel(dst_hbm, zero_hbm, out_hbm, idx_v, ones_v, rbuf, acc_sh, hsem):
    c = lax.axis_index("c")
    s = lax.axis_index("s")
    wid = s * NC + c
    pltpu.sync_copy(zero_hbm.at[pl.ds(0, ROWS_T)], rbuf)
    pltpu.sync_copy(rbuf, acc_sh.at[pl.ds(s * ROWS_T, ROWS_T)])
    for k in range(CHUNK // 16):
        ones_v[pl.ds(k * 16, 16)] = jnp.full((16,), 1.0, jnp.float32)
    pltpu.sync_copy(dst_hbm.at[pl.ds(wid * ECH, ECH)], idx_v)
    plsc.subcore_barrier()

    def body(j, carry):
        pltpu.async_copy(ones_v, acc_sh.at[idx_v.at[j]], hsem, add=True)
        return carry

    lax.fori_loop(0, ECH, body, 0)

    def drain(j, carry):
        pltpu.make_async_copy(ones_v, acc_sh.at[idx_v.at[0]], hsem).wait()
        return carry

    lax.fori_loop(0, ECH, drain, 0)
    plsc.subcore_barrier()
    pltpu.sync_copy(acc_sh.at[pl.ds(s * ROWS_T, ROWS_T)], rbuf)
    pltpu.sync_copy(rbuf, out_hbm.at[pl.ds(c * NP + s * ROWS_T, ROWS_T)])


NBUF = 4


def _make_agg(FP):
    @functools.partial(
        pl.kernel,
        out_type=jax.ShapeDtypeStruct((NC * NP, FP), jnp.float32),
        mesh=_sc_mesh(),
        compiler_params=_SC_PARAMS,
        scratch_types=[
            pltpu.VMEM((ECH, CHUNK), jnp.int32),
            pltpu.VMEM((ECH, CHUNK), jnp.int32),
            pltpu.VMEM((NBUF, CHUNK, FP), jnp.float32),
            pltpu.VMEM((ROWS_T, FP), jnp.float32),
            pltpu.VMEM_SHARED((NP, FP), jnp.float32),
            pltpu.SemaphoreType.DMA((NBUF,)),
            pltpu.SemaphoreType.DMA((NBUF,)),
        ],
    )
    def agg(y_hbm, src_hbm, dst_hbm, zero_hbm, out_hbm,
            idxs_v, idxd_v, bufs, rbuf, acc_sh, gsem, ssem):
        c = lax.axis_index("c")
        s = lax.axis_index("s")
        wid = s * NC + c
        pltpu.sync_copy(zero_hbm.at[pl.ds(0, ROWS_T)], rbuf)
        pltpu.sync_copy(rbuf, acc_sh.at[pl.ds(s * ROWS_T, ROWS_T)])
        pltpu.sync_copy(src_hbm.at[pl.ds(wid * ECH, ECH)], idxs_v)
        pltpu.sync_copy(dst_hbm.at[pl.ds(wid * ECH, ECH)], idxd_v)
        plsc.subcore_barrier()

        for b in range(NBUF):
            pltpu.async_copy(y_hbm.at[idxs_v.at[b]], bufs.at[b], gsem.at[b])

        def group(g, carry):
            for b in range(NBUF):
                jprev = (g - 1) * NBUF + b
                pltpu.make_async_copy(
                    y_hbm.at[idxs_v.at[0]], bufs.at[b], gsem.at[b]).wait()
                pltpu.async_copy(
                    bufs.at[b], acc_sh.at[idxd_v.at[jprev]], ssem.at[b], add=True)
            for b in range(NBUF):
                j = g * NBUF + b
                pltpu.make_async_copy(
                    bufs.at[b], acc_sh.at[idxd_v.at[0]], ssem.at[b]).wait()
                pltpu.async_copy(y_hbm.at[idxs_v.at[j]], bufs.at[b], gsem.at[b])
            return carry

        lax.fori_loop(1, ECH // NBUF, group, 0)

        for b in range(NBUF):
            jprev = ECH - NBUF + b
            pltpu.make_async_copy(
                y_hbm.at[idxs_v.at[0]], bufs.at[b], gsem.at[b]).wait()
            pltpu.async_copy(
                bufs.at[b], acc_sh.at[idxd_v.at[jprev]], ssem.at[b], add=True)
        for b in range(NBUF):
            pltpu.make_async_copy(
                bufs.at[b], acc_sh.at[idxd_v.at[0]], ssem.at[b]).wait()

        plsc.subcore_barrier()
        pltpu.sync_copy(acc_sh.at[pl.ds(s * ROWS_T, ROWS_T)], rbuf)
        pltpu.sync_copy(rbuf, out_hbm.at[pl.ds(c * NP + s * ROWS_T, ROWS_T)])

    return agg


_agg8 = _make_agg(8)
_agg16 = _make_agg(16)
_agg32 = _make_agg(32)


@functools.partial(
    pl.kernel,
    out_type=jax.ShapeDtypeStruct((NW * SCH, PCHUNK), jnp.float32),
    mesh=_sc_mesh(),
    compiler_params=_SC_PARAMS,
    scratch_types=[
        pltpu.VMEM((SCH, PCHUNK), jnp.int32),
        pltpu.VMEM((SCH, PCHUNK), jnp.int32),
        pltpu.VMEM((SCH, PCHUNK), jnp.float32),
        pltpu.VMEM((SCH, PCHUNK), jnp.float32),
        pltpu.SemaphoreType.DMA,
        pltpu.SemaphoreType.DMA,
    ],
)
def _pairs_kernel(p_hbm, q_hbm, si_hbm, sj_hbm, out_hbm,
                  si_v, sj_v, pv, qv, sem1, sem2):
    c = lax.axis_index("c")
    s = lax.axis_index("s")
    wid = s * NC + c
    pltpu.sync_copy(si_hbm.at[pl.ds(wid * SCH, SCH)], si_v)
    pltpu.sync_copy(sj_hbm.at[pl.ds(wid * SCH, SCH)], sj_v)

    def gbody(j, carry):
        pltpu.async_copy(p_hbm.at[si_v.at[j]], pv.at[j], sem1)
        pltpu.async_copy(q_hbm.at[sj_v.at[j]], qv.at[j], sem2)
        return carry

    lax.fori_loop(0, SCH, gbody, 0)
    pltpu.make_async_copy(out_hbm.at[pl.ds(wid * SCH, SCH)], pv, sem1).wait()
    pltpu.make_async_copy(out_hbm.at[pl.ds(wid * SCH, SCH)], qv, sem2).wait()

    def cbody(j, carry):
        for k in range(PCHUNK // 16):
            z = pv[j, pl.ds(k * 16, 16)] + qv[j, pl.ds(k * 16, 16)]
            pv[j, pl.ds(k * 16, 16)] = 1.0 / (1.0 + jnp.exp(-z))
        return carry

    lax.fori_loop(0, SCH, cbody, 0)
    pltpu.sync_copy(pv, out_hbm.at[pl.ds(wid * SCH, SCH)])


def _tck0_body(degt_ref, x_ref, w1_ref, dinv_ref, y1_ref):
    dsum = degt_ref[:, 0:1] + degt_ref[:, 1:2] + 1.0
    rows = lax.broadcasted_iota(jnp.int32, (NP, 1), 0)
    dinv = jnp.where(rows < N, lax.rsqrt(dsum), 0.0)
    dinv_ref[...] = dinv
    y1 = jnp.dot(x_ref[...], w1_ref[...], preferred_element_type=jnp.float32)
    y1_ref[...] = y1 * dinv


def _tck1_body(acc_ref, y1_ref, b1_ref, w2_ref, dinv_ref, y2_ref):
    a = acc_ref[0:NP, :] + acc_ref[NP:2 * NP, :] + y1_ref[...]
    dinv = dinv_ref[...]
    h1 = jnp.maximum(a * dinv + b1_ref[...], 0.0)
    y2_ref[...] = jnp.dot(h1, w2_ref[...], preferred_element_type=jnp.float32) * dinv


def _tck2_body(acc_ref, y2_ref, b2_ref, w3_ref, dinv_ref, y3_ref):
    a = acc_ref[0:NP, :] + acc_ref[NP:2 * NP, :] + y2_ref[...]
    dinv = dinv_ref[...]
    h2 = jnp.maximum(a * dinv + b2_ref[...], 0.0)
    y3_ref[...] = jnp.dot(h2, w3_ref[...], preferred_element_type=jnp.float32) * dinv


def _tck3_body(acc_ref, y3_ref, b3_ref, fcwa_ref, fcwb_ref, fcb_ref, dinv_ref,
               p_ref, q_ref):
    a = acc_ref[0:NP, :] + acc_ref[NP:2 * NP, :] + y3_ref[...]
    h3 = a * dinv_ref[...] + b3_ref[...]
    p_ref[...] = jnp.dot(h3, fcwa_ref[...], preferred_element_type=jnp.float32) + fcb_ref[...]
    q_ref[...] = jnp.dot(h3, fcwb_ref[...], preferred_element_type=jnp.float32)


def _f32(shape):
    return jax.ShapeDtypeStruct(shape, jnp.float32)


def kernel(x, edge_index, samples, W1, b1, W2, b2, W3, b3, fcW, fcb):
    f32 = jnp.float32
    i32 = jnp.int32

    xp = jnp.zeros((NP, D), f32).at[:N].set(x)
    ept_real = E // NW
    npad = EPT - ept_real
    pad_src = jnp.full((NW, npad), N, i32)
    pad_dst = (N + (jnp.arange(npad, dtype=i32)[None, :]
                    + 15 * jnp.arange(NW, dtype=i32)[:, None]) % (NP - N))
    src = jnp.concatenate([edge_index[0].reshape(NW, ept_real), pad_src],
                          axis=1).reshape(NW * ECH, CHUNK)
    dst = jnp.concatenate([edge_index[1].reshape(NW, ept_real),
                           pad_dst.astype(i32)],
                          axis=1).reshape(NW * ECH, CHUNK)
    si = jnp.zeros((SP,), i32).at[:S].set(samples[:, 0]).reshape(NW * SCH, PCHUNK)
    sj = jnp.zeros((SP,), i32).at[:S].set(samples[:, 1]).reshape(NW * SCH, PCHUNK)
    z1 = jnp.zeros((ROWS_T,), f32)
    z8 = jnp.zeros((ROWS_T, 8), f32)
    z16 = jnp.zeros((ROWS_T, 16), f32)
    z32 = jnp.zeros((ROWS_T, 32), f32)

    deg = _hist_kernel(dst, z1)
    degt = deg.reshape(NC, NP).T

    dinv, y1 = pl.pallas_call(
        _tck0_body, out_shape=(_f32((NP, 1)), _f32((NP, 8))),
    )(degt, xp, W1)

    acc1 = _agg8(y1, src, dst, z8)
    y2 = pl.pallas_call(_tck1_body, out_shape=_f32((NP, 16)))(
        acc1, y1, b1.reshape(1, 8), W2, dinv)

    acc2 = _agg16(y2, src, dst, z16)
    y3 = pl.pallas_call(_tck2_body, out_shape=_f32((NP, 32)))(
        acc2, y2, b2.reshape(1, 16), W3, dinv)

    acc3 = _agg32(y3, src, dst, z32)
    p, q = pl.pallas_call(_tck3_body, out_shape=(_f32((NP, 1)), _f32((NP, 1))))(
        acc3, y3, b3.reshape(1, 32), fcW[0:32], fcW[32:64], fcb.reshape(1, 1), dinv)

    out = _pairs_kernel(p.reshape(NP), q.reshape(NP), si, sj)
    return out.reshape(SP)[:S]


# speedup vs baseline: 1.1044x; 1.1044x over previous
import functools

import jax
import jax.numpy as jnp
from jax import lax
from jax.experimental import pallas as pl
from jax.experimental.pallas import tpu as pltpu
from jax.experimental.pallas import tpu_sc as plsc

N = 10000
E = 320000
S = 200000
D = 128

NC = 2
NS = 16
NW = NC * NS

NP = 10240
ROWS_T = NP // NS

CHUNK = 512
ECH = 20
EPT = ECH * CHUNK
EP = EPT * NW

PCHUNK = 128
SCH = 49
SPT = SCH * PCHUNK
SP = SPT * NW


def _sc_mesh():
    return plsc.VectorSubcoreMesh(core_axis_name="c", subcore_axis_name="s",
                                  num_cores=NC, num_subcores=NS)


_SC_PARAMS = pltpu.CompilerParams(use_tc_tiling_on_sc=False)


@functools.partial(
    pl.kernel,
    out_type=jax.ShapeDtypeStruct((NC * NP,), jnp.float32),
    mesh=_sc_mesh(),
    compiler_params=_SC_PARAMS,
    scratch_types=[
        pltpu.VMEM((ECH, CHUNK), jnp.int32),
        pltpu.VMEM((CHUNK,), jnp.float32),
        pltpu.VMEM((ROWS_T,), jnp.float32),
        pltpu.VMEM_SHARED((NP,), jnp.float32),
        pltpu.SemaphoreType.DMA,
    ],
)
def _hist_kernel(dst_hbm, zero_hbm, out_hbm, idx_v, ones_v, rbuf, acc_sh, hsem):
    c = lax.axis_index("c")
    s = lax.axis_index("s")
    wid = s * NC + c
    pltpu.sync_copy(zero_hbm.at[pl.ds(0, ROWS_T)], rbuf)
    pltpu.sync_copy(rbuf, acc_sh.at[pl.ds(s * ROWS_T, ROWS_T)])
    for k in range(CHUNK // 16):
        ones_v[pl.ds(k * 16, 16)] = jnp.full((16,), 1.0, jnp.float32)
    pltpu.sync_copy(dst_hbm.at[pl.ds(wid * ECH, ECH)], idx_v)
    plsc.subcore_barrier()

    def body(j, carry):
        pltpu.async_copy(ones_v, acc_sh.at[idx_v.at[j]], hsem, add=True)
        return carry

    lax.fori_loop(0, ECH, body, 0)

    def drain(j, carry):
        pltpu.make_async_copy(ones_v, acc_sh.at[idx_v.at[0]], hsem).wait()
        return carry

    lax.fori_loop(0, ECH, drain, 0)
    plsc.subcore_barrier()
    pltpu.sync_copy(acc_sh.at[pl.ds(s * ROWS_T, ROWS_T)], rbuf)
    pltpu.sync_copy(rbuf, out_hbm.at[pl.ds(c * NP + s * ROWS_T, ROWS_T)])


NBUF = 4


def _make_agg(FP, stage_y=False):
    scratch = [
        pltpu.VMEM((ECH, CHUNK), jnp.int32),
        pltpu.VMEM((ECH, CHUNK), jnp.int32),
        pltpu.VMEM((NBUF, CHUNK, FP), jnp.float32),
        pltpu.VMEM((ROWS_T, FP), jnp.float32),
        pltpu.VMEM_SHARED((NP, FP), jnp.float32),
        pltpu.SemaphoreType.DMA((NBUF,)),
        pltpu.SemaphoreType.DMA((NBUF,)),
    ]
    if stage_y:
        scratch.append(pltpu.VMEM_SHARED((NP, FP), jnp.float32))

    @functools.partial(
        pl.kernel,
        out_type=jax.ShapeDtypeStruct((NC * NP, FP), jnp.float32),
        mesh=_sc_mesh(),
        compiler_params=_SC_PARAMS,
        scratch_types=scratch,
    )
    def agg(y_hbm, src_hbm, dst_hbm, zero_hbm, out_hbm,
            idxs_v, idxd_v, bufs, rbuf, acc_sh, gsem, ssem, *maybe_ysh):
        c = lax.axis_index("c")
        s = lax.axis_index("s")
        wid = s * NC + c
        if stage_y:
            y_tbl = maybe_ysh[0]
            pltpu.sync_copy(y_hbm.at[pl.ds(s * ROWS_T, ROWS_T)], rbuf)
            pltpu.sync_copy(rbuf, y_tbl.at[pl.ds(s * ROWS_T, ROWS_T)])
        else:
            y_tbl = y_hbm
        pltpu.sync_copy(zero_hbm.at[pl.ds(0, ROWS_T)], rbuf)
        pltpu.sync_copy(rbuf, acc_sh.at[pl.ds(s * ROWS_T, ROWS_T)])
        pltpu.sync_copy(src_hbm.at[pl.ds(wid * ECH, ECH)], idxs_v)
        pltpu.sync_copy(dst_hbm.at[pl.ds(wid * ECH, ECH)], idxd_v)
        plsc.subcore_barrier()

        for b in range(NBUF):
            pltpu.async_copy(y_tbl.at[idxs_v.at[b]], bufs.at[b], gsem.at[b])

        def group(g, carry):
            for b in range(NBUF):
                jprev = (g - 1) * NBUF + b
                pltpu.make_async_copy(
                    y_tbl.at[idxs_v.at[0]], bufs.at[b], gsem.at[b]).wait()
                pltpu.async_copy(
                    bufs.at[b], acc_sh.at[idxd_v.at[jprev]], ssem.at[b], add=True)
            for b in range(NBUF):
                j = g * NBUF + b
                pltpu.make_async_copy(
                    bufs.at[b], acc_sh.at[idxd_v.at[0]], ssem.at[b]).wait()
                pltpu.async_copy(y_tbl.at[idxs_v.at[j]], bufs.at[b], gsem.at[b])
            return carry

        lax.fori_loop(1, ECH // NBUF, group, 0)

        for b in range(NBUF):
            jprev = ECH - NBUF + b
            pltpu.make_async_copy(
                y_tbl.at[idxs_v.at[0]], bufs.at[b], gsem.at[b]).wait()
            pltpu.async_copy(
                bufs.at[b], acc_sh.at[idxd_v.at[jprev]], ssem.at[b], add=True)
        for b in range(NBUF):
            pltpu.make_async_copy(
                bufs.at[b], acc_sh.at[idxd_v.at[0]], ssem.at[b]).wait()

        plsc.subcore_barrier()
        pltpu.sync_copy(acc_sh.at[pl.ds(s * ROWS_T, ROWS_T)], rbuf)
        pltpu.sync_copy(rbuf, out_hbm.at[pl.ds(c * NP + s * ROWS_T, ROWS_T)])

    return agg


_agg8 = _make_agg(8, stage_y=True)
_agg16 = _make_agg(16)
_agg32 = _make_agg(32)


@functools.partial(
    pl.kernel,
    out_type=jax.ShapeDtypeStruct((NW * SCH, PCHUNK), jnp.float32),
    mesh=_sc_mesh(),
    compiler_params=_SC_PARAMS,
    scratch_types=[
        pltpu.VMEM((SCH, PCHUNK), jnp.int32),
        pltpu.VMEM((SCH, PCHUNK), jnp.int32),
        pltpu.VMEM((SCH, PCHUNK), jnp.float32),
        pltpu.VMEM((SCH, PCHUNK), jnp.float32),
        pltpu.SemaphoreType.DMA,
        pltpu.SemaphoreType.DMA,
    ],
)
def _pairs_kernel(p_hbm, q_hbm, si_hbm, sj_hbm, out_hbm,
                  si_v, sj_v, pv, qv, sem1, sem2):
    c = lax.axis_index("c")
    s = lax.axis_index("s")
    wid = s * NC + c
    pltpu.sync_copy(si_hbm.at[pl.ds(wid * SCH, SCH)], si_v)
    pltpu.sync_copy(sj_hbm.at[pl.ds(wid * SCH, SCH)], sj_v)

    def gbody(j, carry):
        pltpu.async_copy(p_hbm.at[si_v.at[j]], pv.at[j], sem1)
        pltpu.async_copy(q_hbm.at[sj_v.at[j]], qv.at[j], sem2)
        return carry

    lax.fori_loop(0, SCH, gbody, 0)
    pltpu.make_async_copy(out_hbm.at[pl.ds(wid * SCH, SCH)], pv, sem1).wait()
    pltpu.make_async_copy(out_hbm.at[pl.ds(wid * SCH, SCH)], qv, sem2).wait()

    def cbody(j, carry):
        for k in range(PCHUNK // 16):
            z = pv[j, pl.ds(k * 16, 16)] + qv[j, pl.ds(k * 16, 16)]
            pv[j, pl.ds(k * 16, 16)] = 1.0 / (1.0 + jnp.exp(-z))
        return carry

    lax.fori_loop(0, SCH, cbody, 0)
    pltpu.sync_copy(pv, out_hbm.at[pl.ds(wid * SCH, SCH)])


def _tck0_body(degt_ref, x_ref, w1_ref, dinv_ref, y1_ref):
    dsum = degt_ref[:, 0:1] + degt_ref[:, 1:2] + 1.0
    rows = lax.broadcasted_iota(jnp.int32, (NP, 1), 0)
    dinv = jnp.where(rows < N, lax.rsqrt(dsum), 0.0)
    dinv_ref[...] = dinv
    y1 = jnp.dot(x_ref[...], w1_ref[...], preferred_element_type=jnp.float32)
    y1_ref[...] = y1 * dinv


def _tck1_body(acc_ref, y1_ref, b1_ref, w2_ref, dinv_ref, y2_ref):
    a = acc_ref[0:NP, :] + acc_ref[NP:2 * NP, :] + y1_ref[...]
    dinv = dinv_ref[...]
    h1 = jnp.maximum(a * dinv + b1_ref[...], 0.0)
    y2_ref[...] = jnp.dot(h1, w2_ref[...], preferred_element_type=jnp.float32) * dinv


def _tck2_body(acc_ref, y2_ref, b2_ref, w3_ref, dinv_ref, y3_ref):
    a = acc_ref[0:NP, :] + acc_ref[NP:2 * NP, :] + y2_ref[...]
    dinv = dinv_ref[...]
    h2 = jnp.maximum(a * dinv + b2_ref[...], 0.0)
    y3_ref[...] = jnp.dot(h2, w3_ref[...], preferred_element_type=jnp.float32) * dinv


def _tck3_body(acc_ref, y3_ref, b3_ref, fcwa_ref, fcwb_ref, fcb_ref, dinv_ref,
               p_ref, q_ref):
    a = acc_ref[0:NP, :] + acc_ref[NP:2 * NP, :] + y3_ref[...]
    h3 = a * dinv_ref[...] + b3_ref[...]
    p_ref[...] = jnp.dot(h3, fcwa_ref[...], preferred_element_type=jnp.float32) + fcb_ref[...]
    q_ref[...] = jnp.dot(h3, fcwb_ref[...], preferred_element_type=jnp.float32)


def _f32(shape):
    return jax.ShapeDtypeStruct(shape, jnp.float32)


def kernel(x, edge_index, samples, W1, b1, W2, b2, W3, b3, fcW, fcb):
    f32 = jnp.float32
    i32 = jnp.int32

    xp = jnp.zeros((NP, D), f32).at[:N].set(x)
    ept_real = E // NW
    npad = EPT - ept_real
    pad_src = jnp.full((NW, npad), N, i32)
    pad_dst = (N + (jnp.arange(npad, dtype=i32)[None, :]
                    + 15 * jnp.arange(NW, dtype=i32)[:, None]) % (NP - N))
    src = jnp.concatenate([edge_index[0].reshape(NW, ept_real), pad_src],
                          axis=1).reshape(NW * ECH, CHUNK)
    dst = jnp.concatenate([edge_index[1].reshape(NW, ept_real),
                           pad_dst.astype(i32)],
                          axis=1).reshape(NW * ECH, CHUNK)
    si = jnp.zeros((SP,), i32).at[:S].set(samples[:, 0]).reshape(NW * SCH, PCHUNK)
    sj = jnp.zeros((SP,), i32).at[:S].set(samples[:, 1]).reshape(NW * SCH, PCHUNK)
    z1 = jnp.zeros((ROWS_T,), f32)
    z8 = jnp.zeros((ROWS_T, 8), f32)
    z16 = jnp.zeros((ROWS_T, 16), f32)
    z32 = jnp.zeros((ROWS_T, 32), f32)

    deg = _hist_kernel(dst, z1)
    degt = deg.reshape(NC, NP).T

    dinv, y1 = pl.pallas_call(
        _tck0_body, out_shape=(_f32((NP, 1)), _f32((NP, 8))),
    )(degt, xp, W1)

    acc1 = _agg8(y1, src, dst, z8)
    y2 = pl.pallas_call(_tck1_body, out_shape=_f32((NP, 16)))(
        acc1, y1, b1.reshape(1, 8), W2, dinv)

    acc2 = _agg16(y2, src, dst, z16)
    y3 = pl.pallas_call(_tck2_body, out_shape=_f32((NP, 32)))(
        acc2, y2, b2.reshape(1, 16), W3, dinv)

    acc3 = _agg32(y3, src, dst, z32)
    p, q = pl.pallas_call(_tck3_body, out_shape=(_f32((NP, 1)), _f32((NP, 1))))(
        acc3, y3, b3.reshape(1, 32), fcW[0:32], fcW[32:64], fcb.reshape(1, 1), dinv)

    out = _pairs_kernel(p.reshape(NP), q.reshape(NP), si, sj)
    return out.reshape(SP)[:S]


# speedup vs baseline: 1.2076x; 1.0934x over previous
import functools

import jax
import jax.numpy as jnp
from jax import lax
from jax.experimental import pallas as pl
from jax.experimental.pallas import tpu as pltpu
from jax.experimental.pallas import tpu_sc as plsc

N = 10000
E = 320000
S = 200000
D = 128

NC = 2
NS = 16
NW = NC * NS

NP = 10240
ROWS_T = NP // NS

CHUNK = 512
ECH = 20
EPT = ECH * CHUNK
EP = EPT * NW

PCHUNK = 128
SCH = 49
SPT = SCH * PCHUNK
SP = SPT * NW


def _sc_mesh():
    return plsc.VectorSubcoreMesh(core_axis_name="c", subcore_axis_name="s",
                                  num_cores=NC, num_subcores=NS)


_SC_PARAMS = pltpu.CompilerParams(use_tc_tiling_on_sc=False)


@functools.partial(
    pl.kernel,
    out_type=jax.ShapeDtypeStruct((NC * NP,), jnp.float32),
    mesh=_sc_mesh(),
    compiler_params=_SC_PARAMS,
    scratch_types=[
        pltpu.VMEM((ECH, CHUNK), jnp.int32),
        pltpu.VMEM((CHUNK,), jnp.float32),
        pltpu.VMEM((ROWS_T,), jnp.float32),
        pltpu.VMEM_SHARED((NP,), jnp.float32),
        pltpu.SemaphoreType.DMA,
    ],
)
def _hist_kernel(dst_hbm, zero_hbm, out_hbm, idx_v, ones_v, rbuf, acc_sh, hsem):
    c = lax.axis_index("c")
    s = lax.axis_index("s")
    wid = s * NC + c
    pltpu.sync_copy(zero_hbm.at[pl.ds(0, ROWS_T)], rbuf)
    pltpu.sync_copy(rbuf, acc_sh.at[pl.ds(s * ROWS_T, ROWS_T)])
    for k in range(CHUNK // 16):
        ones_v[pl.ds(k * 16, 16)] = jnp.full((16,), 1.0, jnp.float32)
    pltpu.sync_copy(dst_hbm.at[pl.ds(wid * ECH, ECH)], idx_v)
    plsc.subcore_barrier()

    def body(j, carry):
        pltpu.async_copy(ones_v, acc_sh.at[idx_v.at[j]], hsem, add=True)
        return carry

    lax.fori_loop(0, ECH, body, 0)

    def drain(j, carry):
        pltpu.make_async_copy(ones_v, acc_sh.at[idx_v.at[0]], hsem).wait()
        return carry

    lax.fori_loop(0, ECH, drain, 0)
    plsc.subcore_barrier()
    pltpu.sync_copy(acc_sh.at[pl.ds(s * ROWS_T, ROWS_T)], rbuf)
    pltpu.sync_copy(rbuf, out_hbm.at[pl.ds(c * NP + s * ROWS_T, ROWS_T)])


NBUF = 4


def _make_agg(FP, stage_y=False):
    scratch = [
        pltpu.VMEM((ECH, CHUNK), jnp.int32),
        pltpu.VMEM((ECH, CHUNK), jnp.int32),
        pltpu.VMEM((NBUF, CHUNK, FP), jnp.float32),
        pltpu.VMEM((ROWS_T, FP), jnp.float32),
        pltpu.VMEM_SHARED((NP, FP), jnp.float32),
        pltpu.SemaphoreType.DMA((NBUF,)),
        pltpu.SemaphoreType.DMA((NBUF,)),
    ]
    if stage_y:
        scratch.append(pltpu.VMEM_SHARED((NP, FP), jnp.float32))

    @functools.partial(
        pl.kernel,
        out_type=jax.ShapeDtypeStruct((NC * NP, FP), jnp.float32),
        mesh=_sc_mesh(),
        compiler_params=_SC_PARAMS,
        scratch_types=scratch,
    )
    def agg(y_hbm, src_hbm, dst_hbm, zero_hbm, out_hbm,
            idxs_v, idxd_v, bufs, rbuf, acc_sh, gsem, ssem, *maybe_ysh):
        c = lax.axis_index("c")
        s = lax.axis_index("s")
        wid = s * NC + c
        if stage_y:
            y_tbl = maybe_ysh[0]
            pltpu.sync_copy(y_hbm.at[pl.ds(s * ROWS_T, ROWS_T)], rbuf)
            pltpu.sync_copy(rbuf, y_tbl.at[pl.ds(s * ROWS_T, ROWS_T)])
        else:
            y_tbl = y_hbm
        pltpu.sync_copy(zero_hbm.at[pl.ds(0, ROWS_T)], rbuf)
        pltpu.sync_copy(rbuf, acc_sh.at[pl.ds(s * ROWS_T, ROWS_T)])
        pltpu.sync_copy(src_hbm.at[pl.ds(wid * ECH, ECH)], idxs_v)
        pltpu.sync_copy(dst_hbm.at[pl.ds(wid * ECH, ECH)], idxd_v)
        plsc.subcore_barrier()

        for b in range(NBUF):
            pltpu.async_copy(y_tbl.at[idxs_v.at[b]], bufs.at[b], gsem.at[b])

        def group(g, carry):
            for b in range(NBUF):
                jprev = (g - 1) * NBUF + b
                pltpu.make_async_copy(
                    y_tbl.at[idxs_v.at[0]], bufs.at[b], gsem.at[b]).wait()
                pltpu.async_copy(
                    bufs.at[b], acc_sh.at[idxd_v.at[jprev]], ssem.at[b], add=True)
            for b in range(NBUF):
                j = g * NBUF + b
                pltpu.make_async_copy(
                    bufs.at[b], acc_sh.at[idxd_v.at[0]], ssem.at[b]).wait()
                pltpu.async_copy(y_tbl.at[idxs_v.at[j]], bufs.at[b], gsem.at[b])
            return carry

        lax.fori_loop(1, ECH // NBUF, group, 0)

        for b in range(NBUF):
            jprev = ECH - NBUF + b
            pltpu.make_async_copy(
                y_tbl.at[idxs_v.at[0]], bufs.at[b], gsem.at[b]).wait()
            pltpu.async_copy(
                bufs.at[b], acc_sh.at[idxd_v.at[jprev]], ssem.at[b], add=True)
        for b in range(NBUF):
            pltpu.make_async_copy(
                bufs.at[b], acc_sh.at[idxd_v.at[0]], ssem.at[b]).wait()

        plsc.subcore_barrier()
        pltpu.sync_copy(acc_sh.at[pl.ds(s * ROWS_T, ROWS_T)], rbuf)
        pltpu.sync_copy(rbuf, out_hbm.at[pl.ds(c * NP + s * ROWS_T, ROWS_T)])

    return agg


_agg8 = _make_agg(8, stage_y=True)
_agg16 = _make_agg(16, stage_y=True)
_agg32 = _make_agg(32)


@functools.partial(
    pl.kernel,
    out_type=jax.ShapeDtypeStruct((NW * SCH, PCHUNK), jnp.float32),
    mesh=_sc_mesh(),
    compiler_params=_SC_PARAMS,
    scratch_types=[
        pltpu.VMEM((SCH, PCHUNK), jnp.int32),
        pltpu.VMEM((SCH, PCHUNK), jnp.int32),
        pltpu.VMEM((SCH, PCHUNK), jnp.float32),
        pltpu.VMEM((SCH, PCHUNK), jnp.float32),
        pltpu.SemaphoreType.DMA,
        pltpu.SemaphoreType.DMA,
    ],
)
def _pairs_kernel(p_hbm, q_hbm, si_hbm, sj_hbm, out_hbm,
                  si_v, sj_v, pv, qv, sem1, sem2):
    c = lax.axis_index("c")
    s = lax.axis_index("s")
    wid = s * NC + c
    pltpu.sync_copy(si_hbm.at[pl.ds(wid * SCH, SCH)], si_v)
    pltpu.sync_copy(sj_hbm.at[pl.ds(wid * SCH, SCH)], sj_v)

    def gbody(j, carry):
        pltpu.async_copy(p_hbm.at[si_v.at[j]], pv.at[j], sem1)
        pltpu.async_copy(q_hbm.at[sj_v.at[j]], qv.at[j], sem2)
        return carry

    lax.fori_loop(0, SCH, gbody, 0)
    pltpu.make_async_copy(out_hbm.at[pl.ds(wid * SCH, SCH)], pv, sem1).wait()
    pltpu.make_async_copy(out_hbm.at[pl.ds(wid * SCH, SCH)], qv, sem2).wait()

    def cbody(j, carry):
        for k in range(PCHUNK // 16):
            z = pv[j, pl.ds(k * 16, 16)] + qv[j, pl.ds(k * 16, 16)]
            pv[j, pl.ds(k * 16, 16)] = 1.0 / (1.0 + jnp.exp(-z))
        return carry

    lax.fori_loop(0, SCH, cbody, 0)
    pltpu.sync_copy(pv, out_hbm.at[pl.ds(wid * SCH, SCH)])


def _tck0_body(degt_ref, x_ref, w1_ref, dinv_ref, y1_ref):
    dsum = degt_ref[:, 0:1] + degt_ref[:, 1:2] + 1.0
    rows = lax.broadcasted_iota(jnp.int32, (NP, 1), 0)
    dinv = jnp.where(rows < N, lax.rsqrt(dsum), 0.0)
    dinv_ref[...] = dinv
    y1 = jnp.dot(x_ref[...], w1_ref[...], preferred_element_type=jnp.float32)
    y1_ref[...] = y1 * dinv


def _tck1_body(acc_ref, y1_ref, b1_ref, w2_ref, dinv_ref, y2_ref):
    a = acc_ref[0:NP, :] + acc_ref[NP:2 * NP, :] + y1_ref[...]
    dinv = dinv_ref[...]
    h1 = jnp.maximum(a * dinv + b1_ref[...], 0.0)
    y2_ref[...] = jnp.dot(h1, w2_ref[...], preferred_element_type=jnp.float32) * dinv


def _tck2_body(acc_ref, y2_ref, b2_ref, w3_ref, dinv_ref, y3_ref):
    a = acc_ref[0:NP, :] + acc_ref[NP:2 * NP, :] + y2_ref[...]
    dinv = dinv_ref[...]
    h2 = jnp.maximum(a * dinv + b2_ref[...], 0.0)
    y3_ref[...] = jnp.dot(h2, w3_ref[...], preferred_element_type=jnp.float32) * dinv


def _tck3_body(acc_ref, y3_ref, b3_ref, fcwa_ref, fcwb_ref, fcb_ref, dinv_ref,
               p_ref, q_ref):
    a = acc_ref[0:NP, :] + acc_ref[NP:2 * NP, :] + y3_ref[...]
    h3 = a * dinv_ref[...] + b3_ref[...]
    p_ref[...] = jnp.dot(h3, fcwa_ref[...], preferred_element_type=jnp.float32) + fcb_ref[...]
    q_ref[...] = jnp.dot(h3, fcwb_ref[...], preferred_element_type=jnp.float32)


def _f32(shape):
    return jax.ShapeDtypeStruct(shape, jnp.float32)


def kernel(x, edge_index, samples, W1, b1, W2, b2, W3, b3, fcW, fcb):
    f32 = jnp.float32
    i32 = jnp.int32

    xp = jnp.zeros((NP, D), f32).at[:N].set(x)
    ept_real = E // NW
    npad = EPT - ept_real
    pad_src = jnp.full((NW, npad), N, i32)
    pad_dst = (N + (jnp.arange(npad, dtype=i32)[None, :]
                    + 15 * jnp.arange(NW, dtype=i32)[:, None]) % (NP - N))
    src = jnp.concatenate([edge_index[0].reshape(NW, ept_real), pad_src],
                          axis=1).reshape(NW * ECH, CHUNK)
    dst = jnp.concatenate([edge_index[1].reshape(NW, ept_real),
                           pad_dst.astype(i32)],
                          axis=1).reshape(NW * ECH, CHUNK)
    si = jnp.zeros((SP,), i32).at[:S].set(samples[:, 0]).reshape(NW * SCH, PCHUNK)
    sj = jnp.zeros((SP,), i32).at[:S].set(samples[:, 1]).reshape(NW * SCH, PCHUNK)
    z1 = jnp.zeros((ROWS_T,), f32)
    z8 = jnp.zeros((ROWS_T, 8), f32)
    z16 = jnp.zeros((ROWS_T, 16), f32)
    z32 = jnp.zeros((ROWS_T, 32), f32)

    deg = _hist_kernel(dst, z1)
    degt = deg.reshape(NC, NP).T

    dinv, y1 = pl.pallas_call(
        _tck0_body, out_shape=(_f32((NP, 1)), _f32((NP, 8))),
    )(degt, xp, W1)

    acc1 = _agg8(y1, src, dst, z8)
    y2 = pl.pallas_call(_tck1_body, out_shape=_f32((NP, 16)))(
        acc1, y1, b1.reshape(1, 8), W2, dinv)

    acc2 = _agg16(y2, src, dst, z16)
    y3 = pl.pallas_call(_tck2_body, out_shape=_f32((NP, 32)))(
        acc2, y2, b2.reshape(1, 16), W3, dinv)

    acc3 = _agg32(y3, src, dst, z32)
    p, q = pl.pallas_call(_tck3_body, out_shape=(_f32((NP, 1)), _f32((NP, 1))))(
        acc3, y3, b3.reshape(1, 32), fcW[0:32], fcW[32:64], fcb.reshape(1, 1), dinv)

    out = _pairs_kernel(p.reshape(NP), q.reshape(NP), si, sj)
    return out.reshape(SP)[:S]
